# baseline (speedup 1.0000x reference)
import functools

import jax
import jax.numpy as jnp
from jax.experimental import pallas as pl
from jax.experimental.pallas import tpu as pltpu

_BM = 16384


_NCHUNK = 8


def _fused_kernel(x_ref, w_ref, b_ref, o_ref):
    w = w_ref[...].astype(jnp.bfloat16)
    bias = b_ref[...]
    rows_per_chunk = _BM // _NCHUNK
    for c in range(_NCHUNK):
        sl = pl.ds(c * rows_per_chunk, rows_per_chunk)
        logits = jnp.dot(x_ref[sl, :].astype(jnp.bfloat16), w,
                         preferred_element_type=jnp.float32) + bias
        m = jnp.max(logits, axis=-1, keepdims=True)
        lse = jnp.log(jnp.sum(jnp.exp(logits - m), axis=-1, keepdims=True))
        o_ref[sl, :] = logits - m - lse


@functools.partial(jax.jit, static_argnames=())
def kernel(x, xl, W, b):
    B, T, D = x.shape
    V = W.shape[1]
    rows = B * T
    x2 = x.reshape(rows, D)
    b2 = b.reshape(1, V)
    grid = (rows // _BM,)
    out = pl.pallas_call(
        _fused_kernel,
        grid=grid,
        in_specs=[
            pl.BlockSpec((_BM, D), lambda i: (i, 0)),
            pl.BlockSpec((D, V), lambda i: (0, 0)),
            pl.BlockSpec((1, V), lambda i: (0, 0)),
        ],
        out_specs=pl.BlockSpec((_BM, V), lambda i: (i, 0)),
        out_shape=jax.ShapeDtypeStruct((rows, V), jnp.float32),
        compiler_params=pltpu.CompilerParams(
            dimension_semantics=("parallel",),
            vmem_limit_bytes=100 * 1024 * 1024,
        ),
    )(x2, W, b2)
    return out.reshape(B, T, V)


# speedup vs baseline: 1.0562x; 1.0562x over previous
import functools

import jax
import jax.numpy as jnp
from jax.experimental import pallas as pl
from jax.experimental.pallas import tpu as pltpu

_C = 4096
_NBUF = 6


def _pipe_kernel(x_hbm, w_ref, b_ref, o_hbm, xbuf, obuf, rsem, wsem):
    n = x_hbm.shape[0] // _C
    w = w_ref[...].astype(jnp.bfloat16)
    bias = b_ref[...]

    def read(i, s):
        return pltpu.make_async_copy(
            x_hbm.at[pl.ds(i * _C, _C), :], xbuf.at[s], rsem.at[s])

    def write(i, s):
        return pltpu.make_async_copy(
            obuf.at[s], o_hbm.at[pl.ds(i * _C, _C), :], wsem.at[s])

    for s in range(min(_NBUF, n)):
        read(s, s).start()

    for i in range(n):
        s = i % _NBUF
        read(i, s).wait()
        if i >= _NBUF:
            write(i - _NBUF, s).wait()
        logits = jnp.dot(xbuf[s].astype(jnp.bfloat16), w,
                         preferred_element_type=jnp.float32) + bias
        m = jnp.max(logits, axis=-1, keepdims=True)
        lse = jnp.log(jnp.sum(jnp.exp(logits - m), axis=-1, keepdims=True))
        obuf[s] = logits - m - lse
        write(i, s).start()
        if i + _NBUF < n:
            read(i + _NBUF, s).start()

    for i in range(max(0, n - _NBUF), n):
        write(i, i % _NBUF).wait()


@functools.partial(jax.jit, static_argnames=())
def kernel(x, xl, W, b):
    B, T, D = x.shape
    V = W.shape[1]
    rows = B * T
    x2 = x.reshape(rows, D)
    b2 = b.reshape(1, V)
    out = pl.pallas_call(
        _pipe_kernel,
        in_specs=[
            pl.BlockSpec(memory_space=pltpu.MemorySpace.HBM),
            pl.BlockSpec(memory_space=pltpu.MemorySpace.VMEM),
            pl.BlockSpec(memory_space=pltpu.MemorySpace.VMEM),
        ],
        out_specs=pl.BlockSpec(memory_space=pltpu.MemorySpace.HBM),
        out_shape=jax.ShapeDtypeStruct((rows, V), jnp.float32),
        scratch_shapes=[
            pltpu.VMEM((_NBUF, _C, D), jnp.float32),
            pltpu.VMEM((_NBUF, _C, V), jnp.float32),
            pltpu.SemaphoreType.DMA((_NBUF,)),
            pltpu.SemaphoreType.DMA((_NBUF,)),
        ],
        compiler_params=pltpu.CompilerParams(
            vmem_limit_bytes=100 * 1024 * 1024,
        ),
    )(x2, W, b2)
    return out.reshape(B, T, V)
